# Initial kernel scaffold; baseline (speedup 1.0000x reference)
import jax
import jax.numpy as jnp
from jax.experimental import pallas as pl


def kernel(x, batch, W, query, gamma, beta):
    raise NotImplementedError("write your pallas kernel here")


# speedup vs baseline: 17.6142x; 17.6142x over previous
import functools

import jax
import jax.numpy as jnp
from jax import lax
from jax.experimental import pallas as pl
from jax.experimental.pallas import tpu as pltpu
from jax.experimental.pallas import tpu_sc as plsc

N = 320000
IN_DIM = 128
HEADS = 4
ATT_DIM = 64
NSEG = 10000
EPAD = 16

SW = 16 * (IN_DIM // 16) * HEADS
DW = EPAD

NWORK = 32
CHUNK = N // NWORK
T = 400
K = 32
NV = IN_DIM // 16

RA = 2000
RC = 2000


def _scores_body(x_ref, w_ref, q_ref, e_ref, v_scr):
    @pl.when(pl.program_id(0) == 0)
    def _():
        v_scr[...] = lax.dot_general(
            q_ref[...], w_ref[...], (((1,), (0,)), ((), ())),
            preferred_element_type=jnp.float32)

    att = lax.dot_general(
        x_ref[...], v_scr[...], (((1,), (1,)), ((), ())),
        preferred_element_type=jnp.float32)
    col = lax.broadcasted_iota(jnp.int32, (RA, EPAD), 1)
    e_ref[...] = jnp.where(col < HEADS, jnp.exp(att), 0.0)


_scores = pl.pallas_call(
    _scores_body,
    grid=(N // RA,),
    in_specs=[
        pl.BlockSpec((RA, IN_DIM), lambda i: (i, 0)),
        pl.BlockSpec((HEADS * ATT_DIM, IN_DIM), lambda i: (0, 0)),
        pl.BlockSpec((EPAD, HEADS * ATT_DIM), lambda i: (0, 0)),
    ],
    out_specs=pl.BlockSpec((RA, EPAD), lambda i: (i, 0)),
    out_shape=jax.ShapeDtypeStruct((N, EPAD), jnp.float32),
    scratch_shapes=[pltpu.VMEM((EPAD, IN_DIM), jnp.float32)],
)


def _segsum_body(x_hbm, e_hbm, b_hbm, s_hbm, d_hbm,
                 xt, et, bt, sstage, dstage, pbuf):
    wid = lax.axis_index("s") * 2 + lax.axis_index("c")
    start = wid * CHUNK
    end = start + CHUNK
    zv = jnp.zeros((16,), jnp.float32)

    def _zero_staging():
        @pl.loop(0, K)
        def _zrow(rr):
            for c in range(NV * HEADS):
                sstage[pl.ds(pl.multiple_of(SW * rr + 16 * c, 16), 16)] = zv
            dstage[pl.ds(pl.multiple_of(DW * rr, 16), 16)] = zv

    def _spills(n, fb):
        @pl.loop(0, n, init_carry=fb)
        def _sp(s_, f):
            pltpu.sync_copy(sstage, s_hbm.at[pl.ds(pl.multiple_of(SW * f, 16),
                                                   SW * K)])
            pltpu.sync_copy(dstage, d_hbm.at[pl.ds(pl.multiple_of(DW * f, 16),
                                                   DW * K)])
            _zero_staging()
            return f + K
        return _sp

    def _write_slot(slot, dvec, accs):
        for c in range(NV * HEADS):
            sstage[pl.ds(pl.multiple_of(SW * slot + 16 * c, 16), 16)] = accs[c]
        dstage[pl.ds(pl.multiple_of(DW * slot, 16), 16)] = dvec

    _zero_staging()

    @pl.when(wid > 0)
    def _():
        pltpu.sync_copy(b_hbm.at[pl.ds(pl.multiple_of(start - 16, 8), 16)], pbuf)
    p = jnp.where(wid > 0, pbuf[...][15], jnp.int32(-1))

    ntiles = (N - start) // T
    init = (p, jnp.bool_(False), jnp.int32(0), jnp.bool_(False),
            jnp.int32(NSEG), zv) + tuple(zv for _ in range(NV * HEADS))

    @pl.loop(0, ntiles, init_carry=init)
    def _tiles(t, tcarry):
        pos = start + t * T
        active = jnp.logical_not(tcarry[3])

        @pl.when(active)
        def _():
            pltpu.sync_copy(
                x_hbm.at[pl.ds(pl.multiple_of(IN_DIM * pos, 16), IN_DIM * T)], xt)
            pltpu.sync_copy(
                e_hbm.at[pl.ds(pl.multiple_of(EPAD * pos, 16), EPAD * T)], et)
            pltpu.sync_copy(b_hbm.at[pl.ds(pl.multiple_of(pos, 8), T)],
                            bt.at[pl.ds(0, T)])

        trip = jnp.where(active, T, 0)

        @pl.loop(0, trip, init_carry=tcarry)
        def _rows(rr, carry):
            cur, started, fb, done, peek = carry[:5]
            dvec = carry[5]
            accs = carry[6:]
            i = pos + rr
            rid = bt[pl.ds(rr, 16)][0]
            fresh = rid != cur
            stop = jnp.logical_and(i >= end, fresh)
            done2 = jnp.logical_or(done, stop)
            process = jnp.logical_and(jnp.logical_not(done2),
                                      jnp.logical_or(started, fresh))
            do_fin = jnp.logical_and(jnp.logical_and(process, fresh), started)

            nspill = jnp.where(do_fin, (cur - fb) // K, 0)
            fb1 = _spills(nspill, fb)

            @pl.when(do_fin)
            def _():
                _write_slot(cur - fb1, dvec, accs)

            ev = et[pl.ds(pl.multiple_of(EPAD * rr, 16), 16)]
            gain = jnp.where(process, 1.0, 0.0)
            keepf = jnp.where(jnp.logical_and(process, fresh), 0.0, 1.0)
            es = tuple(ev[h] * gain for h in range(HEADS))
            xv = tuple(xt[pl.ds(pl.multiple_of(IN_DIM * rr + 16 * j, 16), 16)]
                       for j in range(NV))
            accs2 = tuple(accs[h * NV + j] * keepf + es[h] * xv[j]
                          for h in range(HEADS) for j in range(NV))
            dvec2 = dvec * keepf + ev * gain

            first = jnp.logical_and(process, jnp.logical_not(started))
            fb2 = jnp.where(first,
                            jnp.where(wid == 0, jnp.int32(0), rid), fb1)
            cur2 = jnp.where(process, rid, cur)
            started2 = jnp.logical_or(started, process)
            peek2 = jnp.where(done, peek, jnp.where(stop, rid, peek))
            return (cur2, started2, fb2, done2, peek2, dvec2) + accs2

        return _rows

    cur_f, started_f, fb_f, done_f, peek_f = _tiles[:5]
    dvec_f = _tiles[5]
    accs_f = _tiles[6:]
    peek = jnp.where(done_f, peek_f, jnp.int32(NSEG))

    @pl.when(started_f)
    def _():
        fb2 = _spills((cur_f - fb_f) // K, fb_f)
        _write_slot(cur_f - fb2, dvec_f, accs_f)
        fb3 = _spills((peek - fb2) // K, fb2)
        rem = peek - fb3
        n8 = rem // 8

        @pl.loop(0, n8)
        def _c8(o8):
            o = o8 * 8
            pltpu.sync_copy(
                sstage.at[pl.ds(pl.multiple_of(SW * o, 16), SW * 8)],
                s_hbm.at[pl.ds(pl.multiple_of(SW * (fb3 + o), 16), SW * 8)])
            pltpu.sync_copy(
                dstage.at[pl.ds(pl.multiple_of(DW * o, 16), DW * 8)],
                d_hbm.at[pl.ds(pl.multiple_of(DW * (fb3 + o), 16), DW * 8)])

        @pl.loop(n8 * 8, rem)
        def _c1(o):
            pltpu.sync_copy(
                sstage.at[pl.ds(pl.multiple_of(SW * o, 16), SW)],
                s_hbm.at[pl.ds(pl.multiple_of(SW * (fb3 + o), 16), SW)])
            pltpu.sync_copy(
                dstage.at[pl.ds(pl.multiple_of(DW * o, 16), DW)],
                d_hbm.at[pl.ds(pl.multiple_of(DW * (fb3 + o), 16), DW)])


_segsum = pl.kernel(
    _segsum_body,
    out_type=(jax.ShapeDtypeStruct((NSEG * SW,), jnp.float32),
              jax.ShapeDtypeStruct((NSEG * DW,), jnp.float32)),
    mesh=plsc.VectorSubcoreMesh(core_axis_name="c", subcore_axis_name="s"),
    compiler_params=pltpu.CompilerParams(use_tc_tiling_on_sc=False),
    scratch_types=[
        pltpu.VMEM((T * IN_DIM,), jnp.float32),
        pltpu.VMEM((T * EPAD,), jnp.float32),
        pltpu.VMEM((T + 16,), jnp.int32),
        pltpu.VMEM((K * SW,), jnp.float32),
        pltpu.VMEM((K * DW,), jnp.float32),
        pltpu.VMEM((16,), jnp.int32),
    ],
)


def _epilogue_body(s_ref, d_ref, wt_ref, g_ref, b_ref, o_ref):
    d = d_ref[...]
    row16 = lax.broadcasted_iota(jnp.int32, (EPAD, HEADS * IN_DIM), 0)
    colblk = lax.broadcasted_iota(jnp.int32, (EPAD, HEADS * IN_DIM), 1) // IN_DIM
    sel = jnp.where(row16 == colblk, 1.0, 0.0)
    dexp = lax.dot_general(d, sel, (((1,), (0,)), ((), ())),
                           preferred_element_type=jnp.float32)
    safe = jnp.where(dexp > 0.0, dexp, 1.0)
    pn = s_ref[...] / safe
    pooled = lax.dot_general(pn, wt_ref[...], (((1,), (0,)), ((), ())),
                             preferred_element_type=jnp.float32)
    mu = jnp.mean(pooled, axis=1, keepdims=True)
    xc = pooled - mu
    var = jnp.mean(xc * xc, axis=1, keepdims=True)
    o_ref[...] = xc * lax.rsqrt(var + 1e-5) * g_ref[...] + b_ref[...]


_epilogue = pl.pallas_call(
    _epilogue_body,
    grid=(NSEG // RC,),
    in_specs=[
        pl.BlockSpec((RC, HEADS * IN_DIM), lambda i: (i, 0)),
        pl.BlockSpec((RC, EPAD), lambda i: (i, 0)),
        pl.BlockSpec((HEADS * IN_DIM, HEADS * ATT_DIM), lambda i: (0, 0)),
        pl.BlockSpec((1, HEADS * ATT_DIM), lambda i: (0, 0)),
        pl.BlockSpec((1, HEADS * ATT_DIM), lambda i: (0, 0)),
    ],
    out_specs=pl.BlockSpec((RC, HEADS * ATT_DIM), lambda i: (i, 0)),
    out_shape=jax.ShapeDtypeStruct((NSEG, HEADS * ATT_DIM), jnp.float32),
)


def kernel(x, batch, W, query, gamma, beta):
    q = query.reshape(HEADS, ATT_DIM).astype(jnp.float32)
    qrows = [jnp.pad(q[h:h + 1], ((0, 0), (ATT_DIM * h,
                                           HEADS * ATT_DIM - ATT_DIM * (h + 1))))
             for h in range(HEADS)]
    q16 = jnp.concatenate(
        qrows + [jnp.zeros((EPAD - HEADS, HEADS * ATT_DIM), jnp.float32)], axis=0)
    wblocks = [jnp.pad(W[ATT_DIM * h:ATT_DIM * (h + 1), :].T,
                       ((0, 0), (ATT_DIM * h,
                                 HEADS * ATT_DIM - ATT_DIM * (h + 1))))
               for h in range(HEADS)]
    wt = jnp.concatenate(wblocks, axis=0)

    e = _scores(x, W, q16)
    s1, d1 = _segsum(x.reshape(-1), e.reshape(-1), batch)
    s = s1.reshape(NSEG, SW)
    dsum = d1.reshape(NSEG, DW)
    out = _epilogue(s, dsum, wt,
                    gamma.reshape(1, -1), beta.reshape(1, -1))
    return out


# speedup vs baseline: 25.3556x; 1.4395x over previous
import functools

import jax
import jax.numpy as jnp
from jax import lax
from jax.experimental import pallas as pl
from jax.experimental.pallas import tpu as pltpu
from jax.experimental.pallas import tpu_sc as plsc

N = 320000
IN_DIM = 128
HEADS = 4
ATT_DIM = 64
NSEG = 10000
EPAD = 16

SW = 16 * (IN_DIM // 16) * HEADS
DW = EPAD

UNROLL = 4

NWORK = 32
CHUNK = N // NWORK
T = 400
K = 32
NV = IN_DIM // 16

RA = 2000
RC = 2000


def _scores_body(x_ref, w_ref, q_ref, e_ref, v_scr):
    @pl.when(pl.program_id(0) == 0)
    def _():
        v_scr[...] = lax.dot_general(
            q_ref[...], w_ref[...], (((1,), (0,)), ((), ())),
            preferred_element_type=jnp.float32)

    att = lax.dot_general(
        x_ref[...], v_scr[...], (((1,), (1,)), ((), ())),
        preferred_element_type=jnp.float32)
    col = lax.broadcasted_iota(jnp.int32, (RA, EPAD), 1)
    e_ref[...] = jnp.where(col < HEADS, jnp.exp(att), 0.0)


_scores = pl.pallas_call(
    _scores_body,
    grid=(N // RA,),
    in_specs=[
        pl.BlockSpec((RA, IN_DIM), lambda i: (i, 0)),
        pl.BlockSpec((HEADS * ATT_DIM, IN_DIM), lambda i: (0, 0)),
        pl.BlockSpec((EPAD, HEADS * ATT_DIM), lambda i: (0, 0)),
    ],
    out_specs=pl.BlockSpec((RA, EPAD), lambda i: (i, 0)),
    out_shape=jax.ShapeDtypeStruct((N, EPAD), jnp.float32),
    scratch_shapes=[pltpu.VMEM((EPAD, IN_DIM), jnp.float32)],
)


def _segsum_body(x_hbm, e_hbm, b_hbm, s_hbm, d_hbm,
                 xt, et, bt, sstage, dstage, pbuf, fbref):
    wid = lax.axis_index("s") * 2 + lax.axis_index("c")
    start = wid * CHUNK
    end = start + CHUNK
    zv = jnp.zeros((16,), jnp.float32)

    def _zero_staging():
        @pl.loop(0, K)
        def _zrow(rr):
            for c in range(NV * HEADS):
                sstage[pl.ds(pl.multiple_of(SW * rr + 16 * c, 16), 16)] = zv
            dstage[pl.ds(pl.multiple_of(DW * rr, 16), 16)] = zv

    def _spills1(f):
        pltpu.sync_copy(sstage, s_hbm.at[pl.ds(pl.multiple_of(SW * f, 16),
                                               SW * K)])
        pltpu.sync_copy(dstage, d_hbm.at[pl.ds(pl.multiple_of(DW * f, 16),
                                               DW * K)])
        _zero_staging()
        return f + K

    def _spills(n, fb):
        @pl.loop(0, n, init_carry=fb)
        def _sp(s_, f):
            return _spills1(f)
        return _sp

    def _write_slot(slot, dvec, accs):
        for c in range(NV * HEADS):
            sstage[pl.ds(pl.multiple_of(SW * slot + 16 * c, 16), 16)] = accs[c]
        dstage[pl.ds(pl.multiple_of(DW * slot, 16), 16)] = dvec

    _zero_staging()

    @pl.when(wid > 0)
    def _():
        pltpu.sync_copy(b_hbm.at[pl.ds(pl.multiple_of(start - 16, 8), 16)], pbuf)
    p = jnp.where(wid > 0, pbuf[...][15], jnp.int32(-1))

    ntiles = (N - start) // T
    init = (p, jnp.bool_(False), jnp.bool_(False),
            jnp.int32(NSEG), zv) + tuple(zv for _ in range(NV * HEADS))

    @pl.loop(0, ntiles, init_carry=init)
    def _tiles(t, tcarry):
        pos = start + t * T
        active = jnp.logical_not(tcarry[2])

        @pl.when(active)
        def _():
            pltpu.sync_copy(
                x_hbm.at[pl.ds(pl.multiple_of(IN_DIM * pos, 16), IN_DIM * T)], xt)
            pltpu.sync_copy(
                e_hbm.at[pl.ds(pl.multiple_of(EPAD * pos, 16), EPAD * T)], et)
            pltpu.sync_copy(b_hbm.at[pl.ds(pl.multiple_of(pos, 8), T)],
                            bt.at[pl.ds(0, T)])

        trip = jnp.where(active, T // UNROLL, 0)

        def _row(rr, carry):
            cur, started, done, peek = carry[:4]
            dvec = carry[4]
            accs = carry[5:]
            i = pos + rr
            rid = bt[pl.ds(rr, 16)][0]
            fresh = rid != cur
            stop = jnp.logical_and(i >= end, fresh)
            done2 = jnp.logical_or(done, stop)
            process = jnp.logical_and(jnp.logical_not(done2),
                                      jnp.logical_or(started, fresh))
            do_fin = jnp.logical_and(jnp.logical_and(process, fresh), started)

            @pl.when(do_fin)
            def _():
                fb = fbref[0]

                @pl.loop(0, (cur - fb) // K, init_carry=fb)
                def _sp(s_, f):
                    return _spills1(f)

                _write_slot(cur - _sp, dvec, accs)
                fbref[0] = _sp

            first = jnp.logical_and(process, jnp.logical_not(started))

            @pl.when(first)
            def _():
                fbref[0] = jnp.where(wid == 0, jnp.int32(0), rid)

            ev = et[pl.ds(pl.multiple_of(EPAD * rr, 16), 16)]
            gain = jnp.where(process, 1.0, 0.0)
            keepf = jnp.where(jnp.logical_and(process, fresh), 0.0, 1.0)
            es = tuple(ev[h] * gain for h in range(HEADS))
            xv = tuple(xt[pl.ds(pl.multiple_of(IN_DIM * rr + 16 * j, 16), 16)]
                       for j in range(NV))
            accs2 = tuple(accs[h * NV + j] * keepf + es[h] * xv[j]
                          for h in range(HEADS) for j in range(NV))
            dvec2 = dvec * keepf + ev * gain

            cur2 = jnp.where(process, rid, cur)
            started2 = jnp.logical_or(started, process)
            peek2 = jnp.where(done, peek, jnp.where(stop, rid, peek))
            return (cur2, started2, done2, peek2, dvec2) + accs2

        @pl.loop(0, trip, init_carry=tcarry)
        def _rows(rg, gcarry):
            for u in range(UNROLL):
                gcarry = _row(rg * UNROLL + u, gcarry)
            return gcarry

        return _rows

    cur_f, started_f, done_f, peek_f = _tiles[:4]
    dvec_f = _tiles[4]
    accs_f = _tiles[5:]
    fb_f = fbref[0]
    peek = jnp.where(done_f, peek_f, jnp.int32(NSEG))

    @pl.when(started_f)
    def _():
        fb2 = _spills((cur_f - fb_f) // K, fb_f)
        _write_slot(cur_f - fb2, dvec_f, accs_f)
        fb3 = _spills((peek - fb2) // K, fb2)
        rem = peek - fb3
        n8 = rem // 8

        @pl.loop(0, n8)
        def _c8(o8):
            o = o8 * 8
            pltpu.sync_copy(
                sstage.at[pl.ds(pl.multiple_of(SW * o, 16), SW * 8)],
                s_hbm.at[pl.ds(pl.multiple_of(SW * (fb3 + o), 16), SW * 8)])
            pltpu.sync_copy(
                dstage.at[pl.ds(pl.multiple_of(DW * o, 16), DW * 8)],
                d_hbm.at[pl.ds(pl.multiple_of(DW * (fb3 + o), 16), DW * 8)])

        @pl.loop(n8 * 8, rem)
        def _c1(o):
            pltpu.sync_copy(
                sstage.at[pl.ds(pl.multiple_of(SW * o, 16), SW)],
                s_hbm.at[pl.ds(pl.multiple_of(SW * (fb3 + o), 16), SW)])
            pltpu.sync_copy(
                dstage.at[pl.ds(pl.multiple_of(DW * o, 16), DW)],
                d_hbm.at[pl.ds(pl.multiple_of(DW * (fb3 + o), 16), DW)])


_segsum = pl.kernel(
    _segsum_body,
    out_type=(jax.ShapeDtypeStruct((NSEG * SW,), jnp.float32),
              jax.ShapeDtypeStruct((NSEG * DW,), jnp.float32)),
    mesh=plsc.VectorSubcoreMesh(core_axis_name="c", subcore_axis_name="s"),
    compiler_params=pltpu.CompilerParams(use_tc_tiling_on_sc=False),
    scratch_types=[
        pltpu.VMEM((T * IN_DIM,), jnp.float32),
        pltpu.VMEM((T * EPAD,), jnp.float32),
        pltpu.VMEM((T + 16,), jnp.int32),
        pltpu.VMEM((K * SW,), jnp.float32),
        pltpu.VMEM((K * DW,), jnp.float32),
        pltpu.VMEM((16,), jnp.int32),
        pltpu.SMEM((1,), jnp.int32),
    ],
)


def _epilogue_body(s_ref, d_ref, wt_ref, g_ref, b_ref, o_ref):
    d = d_ref[...]
    row16 = lax.broadcasted_iota(jnp.int32, (EPAD, HEADS * IN_DIM), 0)
    colblk = lax.broadcasted_iota(jnp.int32, (EPAD, HEADS * IN_DIM), 1) // IN_DIM
    sel = jnp.where(row16 == colblk, 1.0, 0.0)
    dexp = lax.dot_general(d, sel, (((1,), (0,)), ((), ())),
                           preferred_element_type=jnp.float32)
    safe = jnp.where(dexp > 0.0, dexp, 1.0)
    pn = s_ref[...] / safe
    pooled = lax.dot_general(pn, wt_ref[...], (((1,), (0,)), ((), ())),
                             preferred_element_type=jnp.float32)
    mu = jnp.mean(pooled, axis=1, keepdims=True)
    xc = pooled - mu
    var = jnp.mean(xc * xc, axis=1, keepdims=True)
    o_ref[...] = xc * lax.rsqrt(var + 1e-5) * g_ref[...] + b_ref[...]


_epilogue = pl.pallas_call(
    _epilogue_body,
    grid=(NSEG // RC,),
    in_specs=[
        pl.BlockSpec((RC, HEADS * IN_DIM), lambda i: (i, 0)),
        pl.BlockSpec((RC, EPAD), lambda i: (i, 0)),
        pl.BlockSpec((HEADS * IN_DIM, HEADS * ATT_DIM), lambda i: (0, 0)),
        pl.BlockSpec((1, HEADS * ATT_DIM), lambda i: (0, 0)),
        pl.BlockSpec((1, HEADS * ATT_DIM), lambda i: (0, 0)),
    ],
    out_specs=pl.BlockSpec((RC, HEADS * ATT_DIM), lambda i: (i, 0)),
    out_shape=jax.ShapeDtypeStruct((NSEG, HEADS * ATT_DIM), jnp.float32),
)


def kernel(x, batch, W, query, gamma, beta):
    q = query.reshape(HEADS, ATT_DIM).astype(jnp.float32)
    qrows = [jnp.pad(q[h:h + 1], ((0, 0), (ATT_DIM * h,
                                           HEADS * ATT_DIM - ATT_DIM * (h + 1))))
             for h in range(HEADS)]
    q16 = jnp.concatenate(
        qrows + [jnp.zeros((EPAD - HEADS, HEADS * ATT_DIM), jnp.float32)], axis=0)
    wblocks = [jnp.pad(W[ATT_DIM * h:ATT_DIM * (h + 1), :].T,
                       ((0, 0), (ATT_DIM * h,
                                 HEADS * ATT_DIM - ATT_DIM * (h + 1))))
               for h in range(HEADS)]
    wt = jnp.concatenate(wblocks, axis=0)

    e = _scores(x, W, q16)
    s1, d1 = _segsum(x.reshape(-1), e.reshape(-1), batch)
    s = s1.reshape(NSEG, SW)
    dsum = d1.reshape(NSEG, DW)
    out = _epilogue(s, dsum, wt,
                    gamma.reshape(1, -1), beta.reshape(1, -1))
    return out


# speedup vs baseline: 25.4104x; 1.0022x over previous
import functools

import jax
import jax.numpy as jnp
from jax import lax
from jax.experimental import pallas as pl
from jax.experimental.pallas import tpu as pltpu
from jax.experimental.pallas import tpu_sc as plsc

N = 320000
IN_DIM = 128
HEADS = 4
ATT_DIM = 64
NSEG = 10000
EPAD = 16

SW = 16 * (IN_DIM // 16) * HEADS
DW = EPAD
EB = 16 * HEADS

UNROLL = 4

NWORK = 32
CHUNK = N // NWORK
T = 400
K = 32
NV = IN_DIM // 16

RA = 2000
RC = 2000


def _scores_body(x_ref, w_ref, q_ref, e_ref, v_scr):
    @pl.when(pl.program_id(0) == 0)
    def _():
        v_scr[...] = lax.dot_general(
            q_ref[...], w_ref[...], (((1,), (0,)), ((), ())),
            preferred_element_type=jnp.float32)

    att = lax.dot_general(
        x_ref[...], v_scr[...], (((1,), (1,)), ((), ())),
        preferred_element_type=jnp.float32)
    col = lax.broadcasted_iota(jnp.int32, (RA, EPAD), 1)
    e = jnp.where(col < HEADS, jnp.exp(att), 0.0)
    rrow = lax.broadcasted_iota(jnp.int32, (EPAD, EB), 0)
    rcol = lax.broadcasted_iota(jnp.int32, (EPAD, EB), 1) // 16
    rep = jnp.where(rrow == rcol, 1.0, 0.0)
    e_ref[...] = lax.dot_general(e, rep, (((1,), (0,)), ((), ())),
                                 preferred_element_type=jnp.float32)


_scores = pl.pallas_call(
    _scores_body,
    grid=(N // RA,),
    in_specs=[
        pl.BlockSpec((RA, IN_DIM), lambda i: (i, 0)),
        pl.BlockSpec((HEADS * ATT_DIM, IN_DIM), lambda i: (0, 0)),
        pl.BlockSpec((EPAD, HEADS * ATT_DIM), lambda i: (0, 0)),
    ],
    out_specs=pl.BlockSpec((RA, EB), lambda i: (i, 0)),
    out_shape=jax.ShapeDtypeStruct((N, EB), jnp.float32),
    scratch_shapes=[pltpu.VMEM((EPAD, IN_DIM), jnp.float32)],
)


def _segsum_body(x_hbm, e_hbm, b_hbm, s_hbm, d_hbm,
                 xt, et, bt, sstage, dstage, pbuf, fbref):
    wid = lax.axis_index("s") * 2 + lax.axis_index("c")
    start = wid * CHUNK
    end = start + CHUNK
    zv = jnp.zeros((16,), jnp.float32)

    def _zero_staging():
        @pl.loop(0, K)
        def _zrow(rr):
            for c in range(NV * HEADS):
                sstage[pl.ds(pl.multiple_of(SW * rr + 16 * c, 16), 16)] = zv
            dstage[pl.ds(pl.multiple_of(DW * rr, 16), 16)] = zv

    def _spills1(f):
        pltpu.sync_copy(sstage, s_hbm.at[pl.ds(pl.multiple_of(SW * f, 16),
                                               SW * K)])
        pltpu.sync_copy(dstage, d_hbm.at[pl.ds(pl.multiple_of(DW * f, 16),
                                               DW * K)])
        _zero_staging()
        return f + K

    def _spills(n, fb):
        @pl.loop(0, n, init_carry=fb)
        def _sp(s_, f):
            return _spills1(f)
        return _sp

    def _write_slot(slot, dvs, accs):
        for c in range(NV * HEADS):
            sstage[pl.ds(pl.multiple_of(SW * slot + 16 * c, 16), 16)] = accs[c]
        lane = lax.iota(jnp.int32, 16)
        dvec = sum((jnp.where(lane == h, 1.0, 0.0) * dvs[h]
                    for h in range(1, HEADS)),
                   jnp.where(lane == 0, 1.0, 0.0) * dvs[0])
        dstage[pl.ds(pl.multiple_of(DW * slot, 16), 16)] = dvec

    _zero_staging()

    @pl.when(wid > 0)
    def _():
        pltpu.sync_copy(b_hbm.at[pl.ds(pl.multiple_of(start - 16, 8), 16)], pbuf)
    p = jnp.where(wid > 0, pbuf[...][15], jnp.int32(-1))

    ntiles = (N - start) // T
    init = (p, jnp.bool_(False), jnp.bool_(False), jnp.int32(NSEG)) \
        + tuple(zv for _ in range(HEADS)) \
        + tuple(zv for _ in range(NV * HEADS))

    @pl.loop(0, ntiles, init_carry=init)
    def _tiles(t, tcarry):
        pos = start + t * T
        active = jnp.logical_not(tcarry[2])

        @pl.when(active)
        def _():
            pltpu.sync_copy(
                x_hbm.at[pl.ds(pl.multiple_of(IN_DIM * pos, 16), IN_DIM * T)], xt)
            pltpu.sync_copy(
                e_hbm.at[pl.ds(pl.multiple_of(EB * pos, 16), EB * T)], et)
            pltpu.sync_copy(b_hbm.at[pl.ds(pl.multiple_of(pos, 8), T)],
                            bt.at[pl.ds(0, T)])

        trip = jnp.where(active, T // UNROLL, 0)

        def _row(rr, carry):
            cur, started, done, peek = carry[:4]
            dvs = carry[4:4 + HEADS]
            accs = carry[4 + HEADS:]
            i = pos + rr
            rid = bt[pl.ds(rr, 16)][0]
            fresh = rid != cur
            stop = jnp.logical_and(i >= end, fresh)
            done2 = jnp.logical_or(done, stop)
            process = jnp.logical_and(jnp.logical_not(done2),
                                      jnp.logical_or(started, fresh))
            do_fin = jnp.logical_and(jnp.logical_and(process, fresh), started)
            first = jnp.logical_and(process, jnp.logical_not(started))

            @pl.when(jnp.logical_or(do_fin, first))
            def _():
                @pl.when(do_fin)
                def _():
                    fb = fbref[0]

                    @pl.loop(0, (cur - fb) // K, init_carry=fb)
                    def _sp(s_, f):
                        return _spills1(f)

                    _write_slot(cur - _sp, dvs, accs)
                    fbref[0] = _sp

                @pl.when(first)
                def _():
                    fbref[0] = jnp.where(wid == 0, jnp.int32(0), rid)

            gain = jnp.where(process, 1.0, 0.0)
            keepf = jnp.where(jnp.logical_and(process, fresh), 0.0, 1.0)
            ebs = tuple(et[pl.ds(pl.multiple_of(EB * rr + 16 * h, 16), 16)]
                        * gain for h in range(HEADS))
            xv = tuple(xt[pl.ds(pl.multiple_of(IN_DIM * rr + 16 * j, 16), 16)]
                       for j in range(NV))
            accs2 = tuple(accs[h * NV + j] * keepf + ebs[h] * xv[j]
                          for h in range(HEADS) for j in range(NV))
            dvs2 = tuple(dvs[h] * keepf + ebs[h] for h in range(HEADS))

            cur2 = jnp.where(process, rid, cur)
            started2 = jnp.logical_or(started, process)
            peek2 = jnp.where(done, peek, jnp.where(stop, rid, peek))
            return (cur2, started2, done2, peek2) + dvs2 + accs2

        @pl.loop(0, trip, init_carry=tcarry)
        def _rows(rg, gcarry):
            for u in range(UNROLL):
                gcarry = _row(rg * UNROLL + u, gcarry)
            return gcarry

        return _rows

    cur_f, started_f, done_f, peek_f = _tiles[:4]
    dvs_f = _tiles[4:4 + HEADS]
    accs_f = _tiles[4 + HEADS:]
    fb_f = fbref[0]
    peek = jnp.where(done_f, peek_f, jnp.int32(NSEG))

    @pl.when(started_f)
    def _():
        fb2 = _spills((cur_f - fb_f) // K, fb_f)
        _write_slot(cur_f - fb2, dvs_f, accs_f)
        fb3 = _spills((peek - fb2) // K, fb2)
        rem = peek - fb3
        n8 = rem // 8

        @pl.loop(0, n8)
        def _c8(o8):
            o = o8 * 8
            pltpu.sync_copy(
                sstage.at[pl.ds(pl.multiple_of(SW * o, 16), SW * 8)],
                s_hbm.at[pl.ds(pl.multiple_of(SW * (fb3 + o), 16), SW * 8)])
            pltpu.sync_copy(
                dstage.at[pl.ds(pl.multiple_of(DW * o, 16), DW * 8)],
                d_hbm.at[pl.ds(pl.multiple_of(DW * (fb3 + o), 16), DW * 8)])

        @pl.loop(n8 * 8, rem)
        def _c1(o):
            pltpu.sync_copy(
                sstage.at[pl.ds(pl.multiple_of(SW * o, 16), SW)],
                s_hbm.at[pl.ds(pl.multiple_of(SW * (fb3 + o), 16), SW)])
            pltpu.sync_copy(
                dstage.at[pl.ds(pl.multiple_of(DW * o, 16), DW)],
                d_hbm.at[pl.ds(pl.multiple_of(DW * (fb3 + o), 16), DW)])


_segsum = pl.kernel(
    _segsum_body,
    out_type=(jax.ShapeDtypeStruct((NSEG * SW,), jnp.float32),
              jax.ShapeDtypeStruct((NSEG * DW,), jnp.float32)),
    mesh=plsc.VectorSubcoreMesh(core_axis_name="c", subcore_axis_name="s"),
    compiler_params=pltpu.CompilerParams(use_tc_tiling_on_sc=False),
    scratch_types=[
        pltpu.VMEM((T * IN_DIM,), jnp.float32),
        pltpu.VMEM((T * EB,), jnp.float32),
        pltpu.VMEM((T + 16,), jnp.int32),
        pltpu.VMEM((K * SW,), jnp.float32),
        pltpu.VMEM((K * DW,), jnp.float32),
        pltpu.VMEM((16,), jnp.int32),
        pltpu.SMEM((1,), jnp.int32),
    ],
)


def _epilogue_body(s_ref, d_ref, wt_ref, g_ref, b_ref, o_ref):
    d = d_ref[...]
    row16 = lax.broadcasted_iota(jnp.int32, (EPAD, HEADS * IN_DIM), 0)
    colblk = lax.broadcasted_iota(jnp.int32, (EPAD, HEADS * IN_DIM), 1) // IN_DIM
    sel = jnp.where(row16 == colblk, 1.0, 0.0)
    dexp = lax.dot_general(d, sel, (((1,), (0,)), ((), ())),
                           preferred_element_type=jnp.float32)
    safe = jnp.where(dexp > 0.0, dexp, 1.0)
    pn = s_ref[...] / safe
    pooled = lax.dot_general(pn, wt_ref[...], (((1,), (0,)), ((), ())),
                             preferred_element_type=jnp.float32)
    mu = jnp.mean(pooled, axis=1, keepdims=True)
    xc = pooled - mu
    var = jnp.mean(xc * xc, axis=1, keepdims=True)
    o_ref[...] = xc * lax.rsqrt(var + 1e-5) * g_ref[...] + b_ref[...]


_epilogue = pl.pallas_call(
    _epilogue_body,
    grid=(NSEG // RC,),
    in_specs=[
        pl.BlockSpec((RC, HEADS * IN_DIM), lambda i: (i, 0)),
        pl.BlockSpec((RC, EPAD), lambda i: (i, 0)),
        pl.BlockSpec((HEADS * IN_DIM, HEADS * ATT_DIM), lambda i: (0, 0)),
        pl.BlockSpec((1, HEADS * ATT_DIM), lambda i: (0, 0)),
        pl.BlockSpec((1, HEADS * ATT_DIM), lambda i: (0, 0)),
    ],
    out_specs=pl.BlockSpec((RC, HEADS * ATT_DIM), lambda i: (i, 0)),
    out_shape=jax.ShapeDtypeStruct((NSEG, HEADS * ATT_DIM), jnp.float32),
)


def kernel(x, batch, W, query, gamma, beta):
    q = query.reshape(HEADS, ATT_DIM).astype(jnp.float32)
    qrows = [jnp.pad(q[h:h + 1], ((0, 0), (ATT_DIM * h,
                                           HEADS * ATT_DIM - ATT_DIM * (h + 1))))
             for h in range(HEADS)]
    q16 = jnp.concatenate(
        qrows + [jnp.zeros((EPAD - HEADS, HEADS * ATT_DIM), jnp.float32)], axis=0)
    wblocks = [jnp.pad(W[ATT_DIM * h:ATT_DIM * (h + 1), :].T,
                       ((0, 0), (ATT_DIM * h,
                                 HEADS * ATT_DIM - ATT_DIM * (h + 1))))
               for h in range(HEADS)]
    wt = jnp.concatenate(wblocks, axis=0)

    e = _scores(x, W, q16)
    s1, d1 = _segsum(x.reshape(-1), e.reshape(-1), batch)
    s = s1.reshape(NSEG, SW)
    dsum = d1.reshape(NSEG, DW)
    out = _epilogue(s, dsum, wt,
                    gamma.reshape(1, -1), beta.reshape(1, -1))
    return out


# speedup vs baseline: 26.3581x; 1.0373x over previous
import functools

import jax
import jax.numpy as jnp
from jax import lax
from jax.experimental import pallas as pl
from jax.experimental.pallas import tpu as pltpu
from jax.experimental.pallas import tpu_sc as plsc

N = 320000
IN_DIM = 128
HEADS = 4
ATT_DIM = 64
NSEG = 10000
EPAD = 16

SW = 16 * (IN_DIM // 16) * HEADS
DW = EPAD
EB = 16 * HEADS

UNROLL = 8

NWORK = 32
CHUNK = N // NWORK
T = 400
K = 32
NV = IN_DIM // 16

RA = 2000
RC = 2000


def _scores_body(x_ref, w_ref, q_ref, e_ref, v_scr):
    @pl.when(pl.program_id(0) == 0)
    def _():
        v_scr[...] = lax.dot_general(
            q_ref[...], w_ref[...], (((1,), (0,)), ((), ())),
            preferred_element_type=jnp.float32)

    att = lax.dot_general(
        x_ref[...], v_scr[...], (((1,), (1,)), ((), ())),
        preferred_element_type=jnp.float32)
    col = lax.broadcasted_iota(jnp.int32, (RA, EPAD), 1)
    e = jnp.where(col < HEADS, jnp.exp(att), 0.0)
    rrow = lax.broadcasted_iota(jnp.int32, (EPAD, EB), 0)
    rcol = lax.broadcasted_iota(jnp.int32, (EPAD, EB), 1) // 16
    rep = jnp.where(rrow == rcol, 1.0, 0.0)
    e_ref[...] = lax.dot_general(e, rep, (((1,), (0,)), ((), ())),
                                 preferred_element_type=jnp.float32)


_scores = pl.pallas_call(
    _scores_body,
    grid=(N // RA,),
    in_specs=[
        pl.BlockSpec((RA, IN_DIM), lambda i: (i, 0)),
        pl.BlockSpec((HEADS * ATT_DIM, IN_DIM), lambda i: (0, 0)),
        pl.BlockSpec((EPAD, HEADS * ATT_DIM), lambda i: (0, 0)),
    ],
    out_specs=pl.BlockSpec((RA, EB), lambda i: (i, 0)),
    out_shape=jax.ShapeDtypeStruct((N, EB), jnp.float32),
    scratch_shapes=[pltpu.VMEM((EPAD, IN_DIM), jnp.float32)],
)


def _segsum_body(x_hbm, e_hbm, b_hbm, s_hbm, d_hbm,
                 xt, et, bt, sstage, dstage, pbuf, fbref):
    wid = lax.axis_index("s") * 2 + lax.axis_index("c")
    start = wid * CHUNK
    end = start + CHUNK
    zv = jnp.zeros((16,), jnp.float32)

    def _zero_staging():
        @pl.loop(0, K)
        def _zrow(rr):
            for c in range(NV * HEADS):
                sstage[pl.ds(pl.multiple_of(SW * rr + 16 * c, 16), 16)] = zv
            dstage[pl.ds(pl.multiple_of(DW * rr, 16), 16)] = zv

    def _spills1(f):
        pltpu.sync_copy(sstage, s_hbm.at[pl.ds(pl.multiple_of(SW * f, 16),
                                               SW * K)])
        pltpu.sync_copy(dstage, d_hbm.at[pl.ds(pl.multiple_of(DW * f, 16),
                                               DW * K)])
        _zero_staging()
        return f + K

    def _spills(n, fb):
        @pl.loop(0, n, init_carry=fb)
        def _sp(s_, f):
            return _spills1(f)
        return _sp

    def _write_slot(slot, dvs, accs):
        for c in range(NV * HEADS):
            sstage[pl.ds(pl.multiple_of(SW * slot + 16 * c, 16), 16)] = accs[c]
        lane = lax.iota(jnp.int32, 16)
        dvec = sum((jnp.where(lane == h, 1.0, 0.0) * dvs[h]
                    for h in range(1, HEADS)),
                   jnp.where(lane == 0, 1.0, 0.0) * dvs[0])
        dstage[pl.ds(pl.multiple_of(DW * slot, 16), 16)] = dvec

    _zero_staging()

    @pl.when(wid > 0)
    def _():
        pltpu.sync_copy(b_hbm.at[pl.ds(pl.multiple_of(start - 16, 8), 16)], pbuf)
    p = jnp.where(wid > 0, pbuf[...][15], jnp.int32(-1))

    ntiles = (N - start) // T
    init = (p, jnp.bool_(False), jnp.bool_(False), jnp.int32(NSEG)) \
        + tuple(zv for _ in range(HEADS)) \
        + tuple(zv for _ in range(NV * HEADS))

    @pl.loop(0, ntiles, init_carry=init)
    def _tiles(t, tcarry):
        pos = start + t * T
        active = jnp.logical_not(tcarry[2])

        @pl.when(active)
        def _():
            pltpu.sync_copy(
                x_hbm.at[pl.ds(pl.multiple_of(IN_DIM * pos, 16), IN_DIM * T)], xt)
            pltpu.sync_copy(
                e_hbm.at[pl.ds(pl.multiple_of(EB * pos, 16), EB * T)], et)
            pltpu.sync_copy(b_hbm.at[pl.ds(pl.multiple_of(pos, 8), T)],
                            bt.at[pl.ds(0, T)])

        trip = jnp.where(active, T // UNROLL, 0)

        def _row(rr, rid, carry):
            cur, started, done, peek = carry[:4]
            dvs = carry[4:4 + HEADS]
            accs = carry[4 + HEADS:]
            i = pos + rr
            fresh = rid != cur
            stop = jnp.logical_and(i >= end, fresh)
            done2 = jnp.logical_or(done, stop)
            process = jnp.logical_and(jnp.logical_not(done2),
                                      jnp.logical_or(started, fresh))
            do_fin = jnp.logical_and(jnp.logical_and(process, fresh), started)
            first = jnp.logical_and(process, jnp.logical_not(started))

            @pl.when(jnp.logical_or(do_fin, first))
            def _():
                @pl.when(do_fin)
                def _():
                    fb = fbref[0]

                    @pl.loop(0, (cur - fb) // K, init_carry=fb)
                    def _sp(s_, f):
                        return _spills1(f)

                    _write_slot(cur - _sp, dvs, accs)
                    fbref[0] = _sp

                @pl.when(first)
                def _():
                    fbref[0] = jnp.where(wid == 0, jnp.int32(0), rid)

            gain = jnp.where(process, 1.0, 0.0)
            keepf = jnp.where(jnp.logical_and(process, fresh), 0.0, 1.0)
            ebs = tuple(et[pl.ds(pl.multiple_of(EB * rr + 16 * h, 16), 16)]
                        * gain for h in range(HEADS))
            xv = tuple(xt[pl.ds(pl.multiple_of(IN_DIM * rr + 16 * j, 16), 16)]
                       for j in range(NV))
            accs2 = tuple(accs[h * NV + j] * keepf + ebs[h] * xv[j]
                          for h in range(HEADS) for j in range(NV))
            dvs2 = tuple(dvs[h] * keepf + ebs[h] for h in range(HEADS))

            cur2 = jnp.where(process, rid, cur)
            started2 = jnp.logical_or(started, process)
            peek2 = jnp.where(done, peek, jnp.where(stop, rid, peek))
            return (cur2, started2, done2, peek2) + dvs2 + accs2

        @pl.loop(0, trip, init_carry=tcarry)
        def _rows(rg, gcarry):
            gids = bt[pl.ds(rg * UNROLL, 16)]
            for u in range(UNROLL):
                gcarry = _row(rg * UNROLL + u, gids[u], gcarry)
            return gcarry

        return _rows

    cur_f, started_f, done_f, peek_f = _tiles[:4]
    dvs_f = _tiles[4:4 + HEADS]
    accs_f = _tiles[4 + HEADS:]
    fb_f = fbref[0]
    peek = jnp.where(done_f, peek_f, jnp.int32(NSEG))

    @pl.when(started_f)
    def _():
        fb2 = _spills((cur_f - fb_f) // K, fb_f)
        _write_slot(cur_f - fb2, dvs_f, accs_f)
        fb3 = _spills((peek - fb2) // K, fb2)
        rem = peek - fb3
        n8 = rem // 8

        @pl.loop(0, n8)
        def _c8(o8):
            o = o8 * 8
            pltpu.sync_copy(
                sstage.at[pl.ds(pl.multiple_of(SW * o, 16), SW * 8)],
                s_hbm.at[pl.ds(pl.multiple_of(SW * (fb3 + o), 16), SW * 8)])
            pltpu.sync_copy(
                dstage.at[pl.ds(pl.multiple_of(DW * o, 16), DW * 8)],
                d_hbm.at[pl.ds(pl.multiple_of(DW * (fb3 + o), 16), DW * 8)])

        @pl.loop(n8 * 8, rem)
        def _c1(o):
            pltpu.sync_copy(
                sstage.at[pl.ds(pl.multiple_of(SW * o, 16), SW)],
                s_hbm.at[pl.ds(pl.multiple_of(SW * (fb3 + o), 16), SW)])
            pltpu.sync_copy(
                dstage.at[pl.ds(pl.multiple_of(DW * o, 16), DW)],
                d_hbm.at[pl.ds(pl.multiple_of(DW * (fb3 + o), 16), DW)])


_segsum = pl.kernel(
    _segsum_body,
    out_type=(jax.ShapeDtypeStruct((NSEG * SW,), jnp.float32),
              jax.ShapeDtypeStruct((NSEG * DW,), jnp.float32)),
    mesh=plsc.VectorSubcoreMesh(core_axis_name="c", subcore_axis_name="s"),
    compiler_params=pltpu.CompilerParams(use_tc_tiling_on_sc=False),
    scratch_types=[
        pltpu.VMEM((T * IN_DIM,), jnp.float32),
        pltpu.VMEM((T * EB,), jnp.float32),
        pltpu.VMEM((T + 16,), jnp.int32),
        pltpu.VMEM((K * SW,), jnp.float32),
        pltpu.VMEM((K * DW,), jnp.float32),
        pltpu.VMEM((16,), jnp.int32),
        pltpu.SMEM((1,), jnp.int32),
    ],
)


def _epilogue_body(s_ref, d_ref, wt_ref, g_ref, b_ref, o_ref):
    d = d_ref[...]
    row16 = lax.broadcasted_iota(jnp.int32, (EPAD, HEADS * IN_DIM), 0)
    colblk = lax.broadcasted_iota(jnp.int32, (EPAD, HEADS * IN_DIM), 1) // IN_DIM
    sel = jnp.where(row16 == colblk, 1.0, 0.0)
    dexp = lax.dot_general(d, sel, (((1,), (0,)), ((), ())),
                           preferred_element_type=jnp.float32)
    safe = jnp.where(dexp > 0.0, dexp, 1.0)
    pn = s_ref[...] / safe
    pooled = lax.dot_general(pn, wt_ref[...], (((1,), (0,)), ((), ())),
                             preferred_element_type=jnp.float32)
    mu = jnp.mean(pooled, axis=1, keepdims=True)
    xc = pooled - mu
    var = jnp.mean(xc * xc, axis=1, keepdims=True)
    o_ref[...] = xc * lax.rsqrt(var + 1e-5) * g_ref[...] + b_ref[...]


_epilogue = pl.pallas_call(
    _epilogue_body,
    grid=(NSEG // RC,),
    in_specs=[
        pl.BlockSpec((RC, HEADS * IN_DIM), lambda i: (i, 0)),
        pl.BlockSpec((RC, EPAD), lambda i: (i, 0)),
        pl.BlockSpec((HEADS * IN_DIM, HEADS * ATT_DIM), lambda i: (0, 0)),
        pl.BlockSpec((1, HEADS * ATT_DIM), lambda i: (0, 0)),
        pl.BlockSpec((1, HEADS * ATT_DIM), lambda i: (0, 0)),
    ],
    out_specs=pl.BlockSpec((RC, HEADS * ATT_DIM), lambda i: (i, 0)),
    out_shape=jax.ShapeDtypeStruct((NSEG, HEADS * ATT_DIM), jnp.float32),
)


def kernel(x, batch, W, query, gamma, beta):
    q = query.reshape(HEADS, ATT_DIM).astype(jnp.float32)
    qrows = [jnp.pad(q[h:h + 1], ((0, 0), (ATT_DIM * h,
                                           HEADS * ATT_DIM - ATT_DIM * (h + 1))))
             for h in range(HEADS)]
    q16 = jnp.concatenate(
        qrows + [jnp.zeros((EPAD - HEADS, HEADS * ATT_DIM), jnp.float32)], axis=0)
    wblocks = [jnp.pad(W[ATT_DIM * h:ATT_DIM * (h + 1), :].T,
                       ((0, 0), (ATT_DIM * h,
                                 HEADS * ATT_DIM - ATT_DIM * (h + 1))))
               for h in range(HEADS)]
    wt = jnp.concatenate(wblocks, axis=0)

    e = _scores(x, W, q16)
    s1, d1 = _segsum(x.reshape(-1), e.reshape(-1), batch)
    s = s1.reshape(NSEG, SW)
    dsum = d1.reshape(NSEG, DW)
    out = _epilogue(s, dsum, wt,
                    gamma.reshape(1, -1), beta.reshape(1, -1))
    return out


# speedup vs baseline: 26.3841x; 1.0010x over previous
import functools

import jax
import jax.numpy as jnp
from jax import lax
from jax.experimental import pallas as pl
from jax.experimental.pallas import tpu as pltpu
from jax.experimental.pallas import tpu_sc as plsc

N = 320000
IN_DIM = 128
HEADS = 4
ATT_DIM = 64
NSEG = 10000
EPAD = 16

SW = 16 * (IN_DIM // 16) * HEADS
DW = EPAD
EB = 16 * HEADS

UNROLL = 8

NWORK = 32
CHUNK = N // NWORK
T = 400
K = 32
NV = IN_DIM // 16

RA = 2000
RC = 2000


def _scores_body(x_ref, w_ref, q_ref, e_ref, v_scr):
    @pl.when(pl.program_id(0) == 0)
    def _():
        v_scr[...] = lax.dot_general(
            q_ref[...], w_ref[...], (((1,), (0,)), ((), ())),
            preferred_element_type=jnp.float32)

    att = lax.dot_general(
        x_ref[...], v_scr[...], (((1,), (1,)), ((), ())),
        preferred_element_type=jnp.float32)
    col = lax.broadcasted_iota(jnp.int32, (RA, EPAD), 1)
    e = jnp.where(col < HEADS, jnp.exp(att), 0.0)
    rrow = lax.broadcasted_iota(jnp.int32, (EPAD, EB), 0)
    rcol = lax.broadcasted_iota(jnp.int32, (EPAD, EB), 1) // 16
    rep = jnp.where(rrow == rcol, 1.0, 0.0)
    e_ref[...] = lax.dot_general(e, rep, (((1,), (0,)), ((), ())),
                                 preferred_element_type=jnp.float32)


_scores = pl.pallas_call(
    _scores_body,
    grid=(N // RA,),
    in_specs=[
        pl.BlockSpec((RA, IN_DIM), lambda i: (i, 0)),
        pl.BlockSpec((HEADS * ATT_DIM, IN_DIM), lambda i: (0, 0)),
        pl.BlockSpec((EPAD, HEADS * ATT_DIM), lambda i: (0, 0)),
    ],
    out_specs=pl.BlockSpec((RA, EB), lambda i: (i, 0)),
    out_shape=jax.ShapeDtypeStruct((N, EB), jnp.float32),
    scratch_shapes=[pltpu.VMEM((EPAD, IN_DIM), jnp.float32)],
)


def _segsum_body(x_hbm, e_hbm, b_hbm, s_hbm, d_hbm,
                 xt, et, bt, sstage, dstage, pbuf, fbref):
    wid = lax.axis_index("s") * 2 + lax.axis_index("c")
    start = wid * CHUNK
    end = start + CHUNK
    zv = jnp.zeros((16,), jnp.float32)

    def _zero_staging():
        @pl.loop(0, K)
        def _zrow(rr):
            for c in range(NV * HEADS):
                sstage[pl.ds(pl.multiple_of(SW * rr + 16 * c, 16), 16)] = zv
            dstage[pl.ds(pl.multiple_of(DW * rr, 16), 16)] = zv

    def _spills1(f):
        pltpu.sync_copy(sstage, s_hbm.at[pl.ds(pl.multiple_of(SW * f, 16),
                                               SW * K)])
        pltpu.sync_copy(dstage, d_hbm.at[pl.ds(pl.multiple_of(DW * f, 16),
                                               DW * K)])
        _zero_staging()
        return f + K

    def _spills(n, fb):
        @pl.loop(0, n, init_carry=fb)
        def _sp(s_, f):
            return _spills1(f)
        return _sp

    def _write_slot(slot, dvs, accs):
        for c in range(NV * HEADS):
            sstage[pl.ds(pl.multiple_of(SW * slot + 16 * c, 16), 16)] = accs[c]
        lane = lax.iota(jnp.int32, 16)
        dvec = sum((jnp.where(lane == h, 1.0, 0.0) * dvs[h]
                    for h in range(1, HEADS)),
                   jnp.where(lane == 0, 1.0, 0.0) * dvs[0])
        dstage[pl.ds(pl.multiple_of(DW * slot, 16), 16)] = dvec

    _zero_staging()

    @pl.when(wid > 0)
    def _():
        pltpu.sync_copy(b_hbm.at[pl.ds(pl.multiple_of(start - 16, 8), 16)], pbuf)
    p = jnp.where(wid > 0, pbuf[...][15], jnp.int32(-1))

    ntiles = (N - start) // T
    init = (p, jnp.bool_(False), jnp.bool_(False), jnp.int32(NSEG)) \
        + tuple(zv for _ in range(HEADS)) \
        + tuple(zv for _ in range(NV * HEADS))

    @pl.loop(0, ntiles, init_carry=init)
    def _tiles(t, tcarry):
        pos = start + t * T
        active = jnp.logical_not(tcarry[2])

        @pl.when(active)
        def _():
            pltpu.sync_copy(x_hbm.at[pl.ds(pl.multiple_of(pos, 8), T)], xt)
            pltpu.sync_copy(e_hbm.at[pl.ds(pl.multiple_of(pos, 8), T)], et)
            pltpu.sync_copy(b_hbm.at[pl.ds(pl.multiple_of(pos, 8), T)],
                            bt.at[pl.ds(0, T)])

        trip = jnp.where(active, T // UNROLL, 0)

        def _row(rr, rid, carry):
            cur, started, done, peek = carry[:4]
            dvs = carry[4:4 + HEADS]
            accs = carry[4 + HEADS:]
            i = pos + rr
            fresh = rid != cur
            stop = jnp.logical_and(i >= end, fresh)
            done2 = jnp.logical_or(done, stop)
            process = jnp.logical_and(jnp.logical_not(done2),
                                      jnp.logical_or(started, fresh))
            do_fin = jnp.logical_and(jnp.logical_and(process, fresh), started)
            first = jnp.logical_and(process, jnp.logical_not(started))

            @pl.when(jnp.logical_or(do_fin, first))
            def _():
                @pl.when(do_fin)
                def _():
                    fb = fbref[0]

                    @pl.loop(0, (cur - fb) // K, init_carry=fb)
                    def _sp(s_, f):
                        return _spills1(f)

                    _write_slot(cur - _sp, dvs, accs)
                    fbref[0] = _sp

                @pl.when(first)
                def _():
                    fbref[0] = jnp.where(wid == 0, jnp.int32(0), rid)

            gain = jnp.where(process, 1.0, 0.0)
            keepf = jnp.where(jnp.logical_and(process, fresh), 0.0, 1.0)
            ebs = tuple(et[rr, pl.ds(16 * h, 16)] * gain for h in range(HEADS))
            xv = tuple(xt[rr, pl.ds(16 * j, 16)] for j in range(NV))
            accs2 = tuple(accs[h * NV + j] * keepf + ebs[h] * xv[j]
                          for h in range(HEADS) for j in range(NV))
            dvs2 = tuple(dvs[h] * keepf + ebs[h] for h in range(HEADS))

            cur2 = jnp.where(process, rid, cur)
            started2 = jnp.logical_or(started, process)
            peek2 = jnp.where(done, peek, jnp.where(stop, rid, peek))
            return (cur2, started2, done2, peek2) + dvs2 + accs2

        @pl.loop(0, trip, init_carry=tcarry)
        def _rows(rg, gcarry):
            gids = bt[pl.ds(rg * UNROLL, 16)]
            for u in range(UNROLL):
                gcarry = _row(rg * UNROLL + u, gids[u], gcarry)
            return gcarry

        return _rows

    cur_f, started_f, done_f, peek_f = _tiles[:4]
    dvs_f = _tiles[4:4 + HEADS]
    accs_f = _tiles[4 + HEADS:]
    fb_f = fbref[0]
    peek = jnp.where(done_f, peek_f, jnp.int32(NSEG))

    @pl.when(started_f)
    def _():
        fb2 = _spills((cur_f - fb_f) // K, fb_f)
        _write_slot(cur_f - fb2, dvs_f, accs_f)
        fb3 = _spills((peek - fb2) // K, fb2)
        rem = peek - fb3
        n8 = rem // 8

        @pl.loop(0, n8)
        def _c8(o8):
            o = o8 * 8
            pltpu.sync_copy(
                sstage.at[pl.ds(pl.multiple_of(SW * o, 16), SW * 8)],
                s_hbm.at[pl.ds(pl.multiple_of(SW * (fb3 + o), 16), SW * 8)])
            pltpu.sync_copy(
                dstage.at[pl.ds(pl.multiple_of(DW * o, 16), DW * 8)],
                d_hbm.at[pl.ds(pl.multiple_of(DW * (fb3 + o), 16), DW * 8)])

        @pl.loop(n8 * 8, rem)
        def _c1(o):
            pltpu.sync_copy(
                sstage.at[pl.ds(pl.multiple_of(SW * o, 16), SW)],
                s_hbm.at[pl.ds(pl.multiple_of(SW * (fb3 + o), 16), SW)])
            pltpu.sync_copy(
                dstage.at[pl.ds(pl.multiple_of(DW * o, 16), DW)],
                d_hbm.at[pl.ds(pl.multiple_of(DW * (fb3 + o), 16), DW)])


_segsum = pl.kernel(
    _segsum_body,
    out_type=(jax.ShapeDtypeStruct((NSEG * SW,), jnp.float32),
              jax.ShapeDtypeStruct((NSEG * DW,), jnp.float32)),
    mesh=plsc.VectorSubcoreMesh(core_axis_name="c", subcore_axis_name="s"),
    compiler_params=pltpu.CompilerParams(use_tc_tiling_on_sc=False),
    scratch_types=[
        pltpu.VMEM((T, IN_DIM), jnp.float32),
        pltpu.VMEM((T, EB), jnp.float32),
        pltpu.VMEM((T + 16,), jnp.int32),
        pltpu.VMEM((K * SW,), jnp.float32),
        pltpu.VMEM((K * DW,), jnp.float32),
        pltpu.VMEM((16,), jnp.int32),
        pltpu.SMEM((1,), jnp.int32),
    ],
)


def _epilogue_body(s_ref, d_ref, wt_ref, g_ref, b_ref, o_ref):
    d = d_ref[...]
    row16 = lax.broadcasted_iota(jnp.int32, (EPAD, HEADS * IN_DIM), 0)
    colblk = lax.broadcasted_iota(jnp.int32, (EPAD, HEADS * IN_DIM), 1) // IN_DIM
    sel = jnp.where(row16 == colblk, 1.0, 0.0)
    dexp = lax.dot_general(d, sel, (((1,), (0,)), ((), ())),
                           preferred_element_type=jnp.float32)
    safe = jnp.where(dexp > 0.0, dexp, 1.0)
    pn = s_ref[...] / safe
    pooled = lax.dot_general(pn, wt_ref[...], (((1,), (0,)), ((), ())),
                             preferred_element_type=jnp.float32)
    mu = jnp.mean(pooled, axis=1, keepdims=True)
    xc = pooled - mu
    var = jnp.mean(xc * xc, axis=1, keepdims=True)
    o_ref[...] = xc * lax.rsqrt(var + 1e-5) * g_ref[...] + b_ref[...]


_epilogue = pl.pallas_call(
    _epilogue_body,
    grid=(NSEG // RC,),
    in_specs=[
        pl.BlockSpec((RC, HEADS * IN_DIM), lambda i: (i, 0)),
        pl.BlockSpec((RC, EPAD), lambda i: (i, 0)),
        pl.BlockSpec((HEADS * IN_DIM, HEADS * ATT_DIM), lambda i: (0, 0)),
        pl.BlockSpec((1, HEADS * ATT_DIM), lambda i: (0, 0)),
        pl.BlockSpec((1, HEADS * ATT_DIM), lambda i: (0, 0)),
    ],
    out_specs=pl.BlockSpec((RC, HEADS * ATT_DIM), lambda i: (i, 0)),
    out_shape=jax.ShapeDtypeStruct((NSEG, HEADS * ATT_DIM), jnp.float32),
)


def kernel(x, batch, W, query, gamma, beta):
    q = query.reshape(HEADS, ATT_DIM).astype(jnp.float32)
    qrows = [jnp.pad(q[h:h + 1], ((0, 0), (ATT_DIM * h,
                                           HEADS * ATT_DIM - ATT_DIM * (h + 1))))
             for h in range(HEADS)]
    q16 = jnp.concatenate(
        qrows + [jnp.zeros((EPAD - HEADS, HEADS * ATT_DIM), jnp.float32)], axis=0)
    wblocks = [jnp.pad(W[ATT_DIM * h:ATT_DIM * (h + 1), :].T,
                       ((0, 0), (ATT_DIM * h,
                                 HEADS * ATT_DIM - ATT_DIM * (h + 1))))
               for h in range(HEADS)]
    wt = jnp.concatenate(wblocks, axis=0)

    e = _scores(x, W, q16)
    s1, d1 = _segsum(x, e, batch)
    s = s1.reshape(NSEG, SW)
    dsum = d1.reshape(NSEG, DW)
    out = _epilogue(s, dsum, wt,
                    gamma.reshape(1, -1), beta.reshape(1, -1))
    return out
